# baseline (speedup 1.0000x reference)
import dataclasses
import functools

import jax
import jax.numpy as jnp
from jax import lax
from jax.experimental import pallas as pl
from jax.experimental.pallas import tpu as pltpu
from jax.experimental.pallas import tpu_sc as plsc

NUM_CLASSES = 80
K = 300
G = 160
NG = 10000
NGH = NG // 2
QCAP = 512
ECAP = 1024
DELTA = 0.01
MCAP = 512


def _sortable(bits):
    return bits ^ (lax.shift_right_arithmetic(bits, 31) & jnp.int32(0x7FFFFFFF))


def _kth_largest_key(keys, k):
    minint = jnp.int32(-2147483648)
    s = _sortable(keys)

    def step(i, p):
        c = p | lax.shift_left(jnp.int32(1), jnp.int32(31) - i)
        cnt = jnp.sum((s >= (c ^ minint)).astype(jnp.int32))
        return jnp.where(cnt >= k, c, p)

    p = lax.fori_loop(0, 32, step, jnp.int32(0))
    return _sortable(p ^ minint)


def _stage1_body(x_ref, gmax_ref, th_ref):
    x = x_ref[0]
    m = jnp.max(x, axis=1)
    gmax_ref[0, 0, :] = m
    kbits = _kth_largest_key(lax.bitcast_convert_type(m, jnp.int32), K)
    t = lax.bitcast_convert_type(kbits, jnp.float32) - DELTA
    th_ref[0, 0, :] = jnp.full((16,), t, dtype=jnp.float32)


def _stage1(x3):
    B = x3.shape[0]
    return pl.pallas_call(
        _stage1_body,
        grid=(B,),
        in_specs=[pl.BlockSpec((1, NG, G), lambda b: (b, 0, 0))],
        out_specs=[pl.BlockSpec((1, 1, NG), lambda b: (b, 0, 0)),
                   pl.BlockSpec((1, 1, 16), lambda b: (b, 0, 0))],
        out_shape=[jax.ShapeDtypeStruct((B, 1, NG), jnp.float32),
                   jax.ShapeDtypeStruct((B, 1, 16), jnp.float32)],
    )(x3)


def _sc_capture(table, gmaxflat, thresh):
    mesh = plsc.VectorSubcoreMesh(core_axis_name="c", subcore_axis_name="s")
    NLOOP = NGH // 16 + 1
    cp = pltpu.CompilerParams()
    if "needs_layout_passes" in pltpu.CompilerParams.__dataclass_fields__:
        cp = dataclasses.replace(cp, needs_layout_passes=False)
    if "use_tc_tiling_on_sc" in pltpu.CompilerParams.__dataclass_fields__:
        cp = dataclasses.replace(cp, use_tc_tiling_on_sc=False)

    @functools.partial(
        pl.kernel, mesh=mesh, compiler_params=cp,
        out_type=[jax.ShapeDtypeStruct((32, ECAP), jnp.float32),
                  jax.ShapeDtypeStruct((32, ECAP), jnp.int32),
                  jax.ShapeDtypeStruct((32, 16), jnp.int32)],
        scratch_types=[pltpu.VMEM((NLOOP * 16,), jnp.float32),
                       pltpu.VMEM((16,), jnp.float32),
                       pltpu.VMEM((QCAP,), jnp.int32),
                       pltpu.VMEM((QCAP, G), jnp.float32),
                       pltpu.VMEM((ECAP,), jnp.float32),
                       pltpu.VMEM((ECAP,), jnp.int32),
                       pltpu.VMEM((16,), jnp.int32),
                       pltpu.SemaphoreType.DMA],
    )
    def sck(table_hbm, gmax_hbm, th_hbm, vals_hbm, fids_hbm, cnt_hbm,
            gm_v, th_v, rows_i, rows_v, ev_v, ef_v, cnt_v, sem):
        wid = lax.axis_index("s") * 2 + lax.axis_index("c")
        b = wid // 2
        h = wid % 2
        gstart = b * NG + h * NGH
        lanes = lax.iota(jnp.int32, 16)

        gm_v[pl.ds(NLOOP * 16 - 16, 16)] = jnp.full((16,), -3e38, jnp.float32)
        pltpu.sync_copy(gmax_hbm.at[pl.ds(gstart, NGH)], gm_v.at[pl.ds(0, NGH)])
        pltpu.sync_copy(th_hbm.at[b], th_v)
        thv = th_v[...]

        @pl.loop(0, QCAP, step=16)
        def _(i):
            rows_i[pl.ds(i, 16)] = jnp.zeros((16,), jnp.int32) + gstart

        def gscan(i, goff):
            v = gm_v[pl.ds(i * 16, 16)]
            m = v >= thv
            plsc.store_compressed(
                rows_i.at[pl.ds(jnp.minimum(goff, QCAP - 16), 16)],
                gstart + i * 16 + lanes, mask=m)
            return goff + jnp.sum(m.astype(jnp.int32))
        gcnt = lax.fori_loop(0, NLOOP, gscan, jnp.int32(0))

        pltpu.async_copy(table_hbm.at[rows_i], rows_v, sem).wait()

        def escan(j, eoff):
            base = plsc.load_gather(rows_i, [jnp.zeros((16,), jnp.int32) + j])
            fidb = (base - b * NG) * G
            def sub(kk, off):
                v = rows_v[j, pl.ds(kk * 16, 16)]
                m = v >= thv
                o = jnp.minimum(off, ECAP - 16)
                plsc.store_compressed(ev_v.at[pl.ds(o, 16)], v, mask=m)
                plsc.store_compressed(ef_v.at[pl.ds(o, 16)],
                                      fidb + kk * 16 + lanes, mask=m)
                return off + jnp.sum(m.astype(jnp.int32))
            for kk in range(G // 16):
                eoff = sub(kk, eoff)
            return eoff
        ecnt = lax.fori_loop(0, gcnt, escan, jnp.int32(0))

        cnt_v[...] = jnp.where(lanes == 0, gcnt,
                               jnp.where(lanes == 1, ecnt, 0))
        pltpu.sync_copy(ev_v, vals_hbm.at[wid])
        pltpu.sync_copy(ef_v, fids_hbm.at[wid])
        pltpu.sync_copy(cnt_v, cnt_hbm.at[wid])

    return sck(table, gmaxflat, thresh)


def _rank_body(s_ref, f_ref, lab_ref, qi_ref, sc_ref):
    E2 = 2 * ECAP
    scr = s_ref[0, :, :]
    fidr = f_ref[0, :, :].astype(jnp.float32)
    keys = lax.bitcast_convert_type(scr, jnp.int32)

    def step(i, p):
        c = p | lax.shift_left(jnp.int32(1), jnp.int32(30) - i)
        cnt = jnp.sum((keys >= c).astype(jnp.int32))
        return jnp.where(cnt >= K, c, p)
    T = lax.fori_loop(0, 31, step, jnp.int32(0))

    m2 = keys >= T
    m2i = m2.astype(jnp.int32)
    x = m2i
    s = 1
    while s < E2:
        x = x + jnp.concatenate(
            [jnp.zeros((1, s), jnp.int32), x[:, :E2 - s]], axis=1)
        s *= 2
    pos = x - m2i
    mtot = jnp.sum(m2i)

    CH = 256
    slot = lax.broadcasted_iota(jnp.int32, (MCAP, CH), 0)
    csc = jnp.zeros((MCAP, 1), jnp.float32)
    cfid = jnp.zeros((MCAP, 1), jnp.float32)
    for c in range(0, E2, CH):
        oh = jnp.where((pos[:, c:c + CH] == slot) & m2[:, c:c + CH], 1.0, 0.0)
        csc = csc + jnp.sum(oh * scr[:, c:c + CH], axis=1, keepdims=True)
        cfid = cfid + jnp.sum(oh * fidr[:, c:c + CH], axis=1, keepdims=True)
    cvalid = lax.broadcasted_iota(jnp.int32, (MCAP, 1), 0) < mtot
    csc = jnp.where(cvalid, csc, -1.0)

    cscr = lax.transpose(csc, (1, 0))
    cfidr = lax.transpose(cfid, (1, 0))

    gt = cscr > csc
    tie = (cscr == csc) & (cfidr < cfid)
    rank = jnp.sum((gt | tie).astype(jnp.int32), axis=1, keepdims=True)
    rankr = lax.transpose(rank, (1, 0))

    KP = 304
    q = lax.broadcasted_iota(jnp.int32, (KP, CH), 0)
    so = jnp.zeros((KP, 1), jnp.float32)
    fo_f = jnp.zeros((KP, 1), jnp.float32)
    for c in range(0, MCAP, CH):
        oh2 = jnp.where(rankr[:, c:c + CH] == q, 1.0, 0.0)
        so = so + jnp.sum(oh2 * cscr[:, c:c + CH], axis=1, keepdims=True)
        fo_f = fo_f + jnp.sum(oh2 * cfidr[:, c:c + CH], axis=1, keepdims=True)
    sot = lax.transpose(so, (1, 0))[0, :K]
    fo = lax.transpose(fo_f, (1, 0))[0, :K].astype(jnp.int32)
    lab_ref[0, 0, :] = fo % NUM_CLASSES
    qi_ref[0, 0, :] = fo // NUM_CLASSES
    sc_ref[0, 0, :] = sot


def _rank(scm, fidr):
    B = scm.shape[0]
    return pl.pallas_call(
        _rank_body,
        grid=(B,),
        in_specs=[pl.BlockSpec((1, 1, 2 * ECAP), lambda b: (b, 0, 0)),
                  pl.BlockSpec((1, 1, 2 * ECAP), lambda b: (b, 0, 0))],
        out_specs=[pl.BlockSpec((1, 1, K), lambda b: (b, 0, 0)),
                   pl.BlockSpec((1, 1, K), lambda b: (b, 0, 0)),
                   pl.BlockSpec((1, 1, K), lambda b: (b, 0, 0))],
        out_shape=[jax.ShapeDtypeStruct((B, 1, K), jnp.int32),
                   jax.ShapeDtypeStruct((B, 1, K), jnp.int32),
                   jax.ShapeDtypeStruct((B, 1, K), jnp.float32)],
    )(scm, fidr)


def kernel(pred_logits, pred_boxes, orig_target_sizes):
    B, N, C = pred_logits.shape
    x3 = pred_logits.reshape(B, NG, G)

    gmax, thresh = _stage1(x3)
    table = pred_logits.reshape(B * NG, G)
    vals, fids, cnts = _sc_capture(table, gmax.reshape(B * NG),
                                   thresh.reshape(B, 16))

    ecnt = cnts[:, 1].reshape(B, 2, 1)
    sc01 = jax.nn.sigmoid(vals).reshape(B, 2, ECAP)
    valid = jnp.arange(ECAP, dtype=jnp.int32)[None, None, :] < ecnt
    scm = jnp.where(valid, sc01, -1.0).reshape(B, 1, 2 * ECAP)

    labels3, qidx3, scout3 = _rank(scm, fids.reshape(B, 1, 2 * ECAP))
    labels = labels3[:, 0, :]
    qidx = qidx3[:, 0, :]
    top_scores = scout3[:, 0, :]

    scale = jnp.tile(orig_target_sizes, (1, 2))[:, None, :]
    cx, cy, w, h = jnp.split(pred_boxes, 4, axis=-1)
    xyxy = jnp.concatenate(
        [cx - 0.5 * w, cy - 0.5 * h, cx + 0.5 * w, cy + 0.5 * h], axis=-1)
    bbox = xyxy * scale
    final_boxes = jnp.take_along_axis(
        bbox, jnp.broadcast_to(qidx[:, :, None], (B, K, 4)), axis=1)
    return (labels, final_boxes, top_scores)
